# Initial kernel scaffold; baseline (speedup 1.0000x reference)
import jax
import jax.numpy as jnp
from jax.experimental import pallas as pl


def kernel(x, conv1_w, conv1_b, conv2_w, conv2_b, fc1_w, fc1_b, fc2_w, fc2_b):
    raise NotImplementedError("write your pallas kernel here")


# speedup vs baseline: 1.0414x; 1.0414x over previous
import jax
import jax.numpy as jnp
from jax.experimental import pallas as pl
from jax.experimental.pallas import tpu as pltpu

B_CONV = 8
B_MLP = 512


def _conv_stage_kernel(x_ref, w1_ref, b1_ref, w2_ref, b2_ref, o_ref,
                       buf1_ref, h1p_ref, buf2_ref):
    B = x_ref.shape[0]

    acc = jnp.zeros((B, 28, 28, 32), jnp.float32)
    for dy in range(3):
        for dx in range(3):
            patch = x_ref[:, dy:dy + 28, dx:dx + 28, :]
            acc = acc + patch * w1_ref[dy, dx]
    buf1_ref[...] = acc

    c0 = buf1_ref[:, :, pl.ds(0, 14, stride=2), :]
    c1 = buf1_ref[:, :, pl.ds(1, 14, stride=2), :]
    rm = jnp.maximum(c0, c1)
    rm = rm.reshape(B, 14, 2, 14, 32)
    h1 = jnp.maximum(rm[:, :, 0], rm[:, :, 1])
    h1 = jnp.maximum(h1 + b1_ref[...], 0.0)

    h1p_ref[...] = jnp.pad(h1, ((0, 0), (1, 4), (1, 1), (0, 0))).reshape(B, 304, 32)

    acc2 = jnp.zeros((B * 256, 32), jnp.float32)
    for dy in range(3):
        for dx in range(3):
            s = dy * 16 + dx
            slab = h1p_ref[:, s:s + 256, :].reshape(B * 256, 32)
            acc2 = acc2 + jnp.dot(slab, w2_ref[dy, dx],
                                  preferred_element_type=jnp.float32)
    buf2_ref[...] = acc2.reshape(B, 16, 16, 32)

    d0 = buf2_ref[:, :, pl.ds(0, 7, stride=2), :]
    d1 = buf2_ref[:, :, pl.ds(1, 7, stride=2), :]
    sm = jnp.maximum(d0, d1)[:, :14]
    sm = sm.reshape(B, 7, 2, 7, 32)
    h2 = jnp.maximum(sm[:, :, 0], sm[:, :, 1])
    h2 = jnp.maximum(h2 + b2_ref[...], 0.0)
    o_ref[...] = h2.reshape(B, 49, 32)


def _conv_stage(xpad, w1, b1, w2, b2):
    N = xpad.shape[0]
    B = B_CONV
    return pl.pallas_call(
        _conv_stage_kernel,
        out_shape=jax.ShapeDtypeStruct((N, 49, 32), jnp.float32),
        grid_spec=pltpu.PrefetchScalarGridSpec(
            num_scalar_prefetch=0,
            grid=(N // B,),
            in_specs=[
                pl.BlockSpec((B, 30, 30, 1), lambda n: (n, 0, 0, 0)),
                pl.BlockSpec((3, 3, 1, 32), lambda n: (0, 0, 0, 0)),
                pl.BlockSpec((1, 32), lambda n: (0, 0)),
                pl.BlockSpec((3, 3, 32, 32), lambda n: (0, 0, 0, 0)),
                pl.BlockSpec((1, 32), lambda n: (0, 0)),
            ],
            out_specs=pl.BlockSpec((B, 49, 32), lambda n: (n, 0, 0)),
            scratch_shapes=[
                pltpu.VMEM((B, 28, 28, 32), jnp.float32),
                pltpu.VMEM((B, 304, 32), jnp.float32),
                pltpu.VMEM((B, 16, 16, 32), jnp.float32),
            ],
        ),
        compiler_params=pltpu.CompilerParams(dimension_semantics=("parallel",)),
    )(xpad, w1, b1.reshape(1, 32), w2, b2.reshape(1, 32))


def _mlp_kernel(x_ref, w1_ref, b1_ref, w2_ref, b2_ref, o_ref):
    h = jnp.dot(x_ref[...], w1_ref[...],
                preferred_element_type=jnp.float32) + b1_ref[...]
    h = jnp.maximum(h, 0.0)
    logits = jnp.dot(h, w2_ref[...],
                     preferred_element_type=jnp.float32) + b2_ref[...]
    m = jnp.max(logits, axis=1, keepdims=True)
    s = logits - m
    lse = jnp.log(jnp.sum(jnp.exp(s), axis=1, keepdims=True))
    o_ref[...] = s - lse


def _mlp_stage(x2d, w1t, b1, w2t, b2):
    N, D = x2d.shape
    H1 = w1t.shape[1]
    C = w2t.shape[1]
    B = B_MLP
    return pl.pallas_call(
        _mlp_kernel,
        out_shape=jax.ShapeDtypeStruct((N, C), jnp.float32),
        grid_spec=pltpu.PrefetchScalarGridSpec(
            num_scalar_prefetch=0,
            grid=(N // B,),
            in_specs=[
                pl.BlockSpec((B, D), lambda n: (n, 0)),
                pl.BlockSpec((D, H1), lambda n: (0, 0)),
                pl.BlockSpec((1, H1), lambda n: (0, 0)),
                pl.BlockSpec((H1, C), lambda n: (0, 0)),
                pl.BlockSpec((1, C), lambda n: (0, 0)),
            ],
            out_specs=pl.BlockSpec((B, C), lambda n: (n, 0)),
        ),
        compiler_params=pltpu.CompilerParams(dimension_semantics=("parallel",)),
    )(x2d, w1t, b1.reshape(1, -1), w2t, b2.reshape(1, -1))


def kernel(x, conv1_w, conv1_b, conv2_w, conv2_b, fc1_w, fc1_b, fc2_w, fc2_b):
    N = x.shape[0]
    xn = jnp.transpose(x, (0, 2, 3, 1))
    xpad = jnp.pad(xn, ((0, 0), (1, 1), (1, 1), (0, 0)))

    w1 = jnp.transpose(conv1_w, (2, 3, 1, 0))
    w2 = jnp.transpose(conv2_w, (2, 3, 1, 0))

    h = _conv_stage(xpad, w1, conv1_b, w2, conv2_b)

    hflat = h.reshape(N, 49 * 32)
    idx = jnp.arange(49 * 32)
    nchw_col = (idx % 32) * 49 + idx // 32
    w1t = fc1_w[:, nchw_col].T
    w2t = fc2_w.T
    return _mlp_stage(hflat, w1t, fc1_b, w2t, fc2_b)
